# baseline (speedup 1.0000x reference)
import functools

import jax
import jax.numpy as jnp
from jax import lax
from jax.experimental import pallas as pl
from jax.experimental.pallas import tpu as pltpu
from jax.experimental.pallas import tpu_sc as plsc

F32 = jnp.float32
NC = 2
NS = 16
NW = NC * NS
WIN = 128
LANES = 16
G = 5


def _ceil_to(v, m):
    return (v + m - 1) // m * m


def _sc_mesh():
    return plsc.VectorSubcoreMesh(core_axis_name="c", subcore_axis_name="s")


_SC_PARAMS = pltpu.CompilerParams(use_tc_tiling_on_sc=False,
                                  needs_layout_passes=False)


def _deg_kernel(n1, nwin):
    stripe = n1 // NS

    @functools.partial(
        pl.kernel,
        out_type=jax.ShapeDtypeStruct((NC, n1), F32),
        mesh=_sc_mesh(),
        scratch_types=[
            pltpu.VMEM((nwin, WIN), jnp.int32),
            pltpu.VMEM((WIN,), F32),
            pltpu.VMEM((stripe,), F32),
            pltpu.VMEM_SHARED((n1,), F32),
            pltpu.SemaphoreType.DMA,
        ],
        compiler_params=_SC_PARAMS,
    )
    def deg(dstp_hbm, out_hbm, idx_v, ones_v, zb_v, acc_sh, sem):
        c = lax.axis_index("c")
        s = lax.axis_index("s")
        wid = s * NC + c

        @pl.loop(0, WIN, step=LANES)
        def _(i):
            ones_v[pl.ds(i, LANES)] = jnp.full((LANES,), 1.0, F32)

        @pl.loop(0, stripe, step=LANES)
        def _(i):
            zb_v[pl.ds(i, LANES)] = jnp.zeros((LANES,), F32)

        base = s * stripe
        pltpu.sync_copy(zb_v, acc_sh.at[pl.ds(base, stripe)])
        pltpu.sync_copy(dstp_hbm.at[wid], idx_v)
        plsc.subcore_barrier()

        @pl.loop(0, nwin, step=2 * G)
        def _(j):
            for k in range(2 * G):
                pltpu.async_copy(ones_v, acc_sh.at[idx_v.at[j + k]], sem,
                                 add=True)
            for k in range(2 * G):
                pltpu.make_async_copy(
                    ones_v, acc_sh.at[idx_v.at[j + k]], sem).wait()

        plsc.subcore_barrier()
        pltpu.sync_copy(acc_sh.at[pl.ds(base, stripe)], zb_v)
        pltpu.sync_copy(zb_v, out_hbm.at[c].at[pl.ds(base, stripe)])

    return deg


def _hop_kernel(n2, cp, nwin, second):
    stripe = n2 // NS
    rb = max(G * WIN, stripe)
    ngrp = nwin // G
    npairs = ngrp // 2
    cols = list(range(0, cp - 15, LANES))
    if cols[-1] + LANES < cp:
        cols.append(cp - LANES)

    @functools.partial(
        pl.kernel,
        out_type=jax.ShapeDtypeStruct((NC, n2, cp), F32),
        mesh=_sc_mesh(),
        scratch_types=[
            pltpu.VMEM((nwin, WIN), jnp.int32),
            pltpu.VMEM((nwin, WIN), jnp.int32),
            pltpu.VMEM((rb, cp), F32),
            pltpu.VMEM((rb, cp), F32),
            pltpu.VMEM((stripe,), F32),
            pltpu.VMEM_SHARED((n2, cp), F32),
            pltpu.VMEM_SHARED((n2, cp), F32),
            pltpu.SemaphoreType.DMA,
            pltpu.SemaphoreType.DMA,
            pltpu.SemaphoreType.DMA,
            pltpu.SemaphoreType.DMA,
        ],
        compiler_params=_SC_PARAMS,
    )
    def hop(h_hbm, dv_hbm, srcp_hbm, dstp_hbm, out_hbm, src_v, dst_v, rows_a,
            rows_b, dv_v, u_sh, acc_sh, gsa, gsb, ssa, ssb):
        c = lax.axis_index("c")
        s = lax.axis_index("s")
        wid = s * NC + c
        base = s * stripe

        if second:
            pltpu.async_copy(h_hbm.at[0].at[pl.ds(base, stripe), :],
                             rows_a.at[pl.ds(0, stripe), :], gsa)
            pltpu.async_copy(h_hbm.at[1].at[pl.ds(base, stripe), :],
                             rows_b.at[pl.ds(0, stripe), :], gsb)
        else:
            pltpu.async_copy(h_hbm.at[pl.ds(base, stripe), :],
                             rows_a.at[pl.ds(0, stripe), :], gsa)
        pltpu.sync_copy(srcp_hbm.at[wid], src_v)
        pltpu.sync_copy(dstp_hbm.at[wid], dst_v)
        pltpu.sync_copy(dv_hbm.at[pl.ds(base, stripe)], dv_v)

        if second:
            pltpu.make_async_copy(h_hbm.at[0].at[pl.ds(base, stripe), :],
                                  rows_a.at[pl.ds(0, stripe), :], gsa).wait()
            pltpu.make_async_copy(h_hbm.at[1].at[pl.ds(base, stripe), :],
                                  rows_b.at[pl.ds(0, stripe), :], gsb).wait()
        else:
            pltpu.make_async_copy(h_hbm.at[pl.ds(base, stripe), :],
                                  rows_a.at[pl.ds(0, stripe), :], gsa).wait()

        @pl.loop(0, stripe, step=2)
        def _(r):
            for rr in range(2):
                vs = plsc.load_gather(
                    dv_v, [jnp.full((LANES,), r + rr, jnp.int32)])
                va = [rows_a[r + rr, pl.ds(c0, LANES)] for c0 in cols]
                if second:
                    va = [v + rows_b[r + rr, pl.ds(c0, LANES)]
                          for v, c0 in zip(va, cols)]
                for v, c0 in zip(va, cols):
                    rows_a[r + rr, pl.ds(c0, LANES)] = v * vs

        pltpu.async_copy(rows_a.at[pl.ds(0, stripe), :],
                         u_sh.at[pl.ds(base, stripe), :], gsa)

        @pl.when(c == 0)
        def _():
            pltpu.async_copy(rows_a.at[pl.ds(0, stripe), :],
                             acc_sh.at[pl.ds(base, stripe), :], gsb)

        @pl.when(c != 0)
        def _():
            @pl.loop(0, stripe)
            def _(r):
                for c0 in cols:
                    rows_b[r, pl.ds(c0, LANES)] = jnp.zeros((LANES,), F32)

            pltpu.async_copy(rows_b.at[pl.ds(0, stripe), :],
                             acc_sh.at[pl.ds(base, stripe), :], gsb)

        pltpu.make_async_copy(rows_a.at[pl.ds(0, stripe), :],
                              u_sh.at[pl.ds(base, stripe), :], gsa).wait()
        pltpu.make_async_copy(rows_a.at[pl.ds(0, stripe), :],
                              acc_sh.at[pl.ds(base, stripe), :], gsb).wait()

        def grp_gather(buf, sem, g):
            for k in range(G):
                pltpu.async_copy(u_sh.at[src_v.at[g * G + k]],
                                 buf.at[pl.ds(k * WIN, WIN), :], sem)

        def grp_gather_wait(buf, sem, g):
            for k in range(G):
                pltpu.make_async_copy(u_sh.at[src_v.at[g * G + k]],
                                      buf.at[pl.ds(k * WIN, WIN), :],
                                      sem).wait()

        def grp_scatter(buf, sem, g):
            for k in range(G):
                pltpu.async_copy(buf.at[pl.ds(k * WIN, WIN), :],
                                 acc_sh.at[dst_v.at[g * G + k]], sem,
                                 add=True)

        def grp_scatter_wait(buf, sem, g):
            for k in range(G):
                pltpu.make_async_copy(buf.at[pl.ds(k * WIN, WIN), :],
                                      acc_sh.at[dst_v.at[g * G + k]],
                                      sem).wait()

        plsc.subcore_barrier()
        grp_gather(rows_a, gsa, 0)

        @pl.loop(0, npairs)
        def _(it):
            g = it * 2
            grp_gather(rows_b, gsb, g + 1)
            grp_gather_wait(rows_a, gsa, g)
            grp_scatter(rows_a, ssa, g)
            grp_scatter_wait(rows_a, ssa, g)

            @pl.when(it + 1 < npairs)
            def _():
                grp_gather(rows_a, gsa, g + 2)

            grp_gather_wait(rows_b, gsb, g + 1)
            grp_scatter(rows_b, ssb, g + 1)
            grp_scatter_wait(rows_b, ssb, g + 1)

        plsc.subcore_barrier()
        pltpu.sync_copy(acc_sh.at[pl.ds(base, stripe), :],
                        out_hbm.at[c].at[pl.ds(base, stripe), :])

    return hop


def _matmul(x, w48, n, n2, cp):
    def body(x_ref, w_ref, o_ref):
        o_ref[0:n, :] = jnp.dot(x_ref[...], w_ref[...],
                                preferred_element_type=F32)
        o_ref[n:n2, :] = jnp.zeros((n2 - n, cp), F32)

    return pl.pallas_call(
        body, out_shape=jax.ShapeDtypeStruct((n2, cp), F32))(x, w48)


def _dv_kernel(degp, n1):
    def body(degp_ref, dv1_ref, dv2_ref):
        deg = degp_ref[0:1, :] + degp_ref[1:2, :] + 1.0
        dv1_ref[...] = lax.rsqrt(deg).reshape(n1)
        dv2_ref[...] = (1.0 / deg).reshape(n1)

    return pl.pallas_call(
        body,
        out_shape=(jax.ShapeDtypeStruct((n1,), F32),
                   jax.ShapeDtypeStruct((n1,), F32)))(degp)


def _finalize(h2, dv1, b2, n, c):
    def body(h2_ref, dv1_ref, b_ref, o_ref):
        dinv = dv1_ref[0:n].reshape(n, 1)
        h = (h2_ref[0, 0:n, :] + h2_ref[1, 0:n, :]) * dinv
        logits = h[:, :c] + b_ref[...]
        m = jnp.max(logits, axis=1, keepdims=True)
        e = jnp.exp(logits - m)
        lse = jnp.log(jnp.sum(e, axis=1, keepdims=True)) + m
        o_ref[...] = logits - lse

    return pl.pallas_call(
        body, out_shape=jax.ShapeDtypeStruct((n, c), F32))(h2, dv1, b2)


def kernel(x, edge_index, W, b):
    n, d = x.shape
    e = edge_index.shape[1]
    c = W.shape[1]
    cp = _ceil_to(c, 8)

    ew = _ceil_to(-(-e // NW), 2 * G * WIN)
    nwin = ew // WIN
    ep = NW * ew
    n1 = _ceil_to(n + 16, NS * LANES)
    n2 = _ceil_to(n + 16, NS * 8)

    pad = ep - e
    trash = (n + (jnp.arange(pad, dtype=jnp.int32) % 16)).reshape(1, pad)
    dstp = jnp.concatenate([edge_index[1:2], trash],
                           axis=1).reshape(NW, nwin, WIN)
    srcp = jnp.concatenate([edge_index[0:1], jnp.zeros((1, pad), jnp.int32)],
                           axis=1).reshape(NW, nwin, WIN)

    w48 = jnp.pad(W, ((0, 0), (0, cp - c)))
    b2 = b.reshape(1, c)

    degp = _deg_kernel(n1, nwin)(dstp)
    y = _matmul(x, w48, n, n2, cp)
    dv1, dv2 = _dv_kernel(degp, n1)

    h1 = _hop_kernel(n2, cp, nwin, second=False)(y, dv1, srcp, dstp)
    h2 = _hop_kernel(n2, cp, nwin, second=True)(h1, dv2, srcp, dstp)
    return _finalize(h2, dv1, b2, n, c)


# speedup vs baseline: 1.0285x; 1.0285x over previous
import functools

import jax
import jax.numpy as jnp
from jax import lax
from jax.experimental import pallas as pl
from jax.experimental.pallas import tpu as pltpu
from jax.experimental.pallas import tpu_sc as plsc

F32 = jnp.float32
NC = 2
NS = 16
NW = NC * NS
WIN = 128
LANES = 16
G = 5


def _ceil_to(v, m):
    return (v + m - 1) // m * m


def _sc_mesh():
    return plsc.VectorSubcoreMesh(core_axis_name="c", subcore_axis_name="s")


_SC_PARAMS = pltpu.CompilerParams(use_tc_tiling_on_sc=False,
                                  needs_layout_passes=False)


def _deg_kernel(n1, nwin):
    stripe = n1 // NS

    @functools.partial(
        pl.kernel,
        out_type=jax.ShapeDtypeStruct((NC, n1), F32),
        mesh=_sc_mesh(),
        scratch_types=[
            pltpu.VMEM((nwin, WIN), jnp.int32),
            pltpu.VMEM((WIN,), F32),
            pltpu.VMEM((stripe,), F32),
            pltpu.VMEM_SHARED((n1,), F32),
            pltpu.SemaphoreType.DMA,
        ],
        compiler_params=_SC_PARAMS,
    )
    def deg(eip_hbm, out_hbm, idx_v, ones_v, zb_v, acc_sh, sem):
        c = lax.axis_index("c")
        s = lax.axis_index("s")
        wid = s * NC + c

        @pl.loop(0, WIN, step=LANES)
        def _(i):
            ones_v[pl.ds(i, LANES)] = jnp.full((LANES,), 1.0, F32)

        @pl.loop(0, stripe, step=LANES)
        def _(i):
            zb_v[pl.ds(i, LANES)] = jnp.zeros((LANES,), F32)

        base = s * stripe
        pltpu.sync_copy(zb_v, acc_sh.at[pl.ds(base, stripe)])
        pltpu.sync_copy(eip_hbm.at[1].at[wid], idx_v)
        plsc.subcore_barrier()

        @pl.loop(0, nwin, step=2 * G)
        def _(j):
            for k in range(2 * G):
                pltpu.async_copy(ones_v, acc_sh.at[idx_v.at[j + k]], sem,
                                 add=True)
            for k in range(2 * G):
                pltpu.make_async_copy(
                    ones_v, acc_sh.at[idx_v.at[j + k]], sem).wait()

        plsc.subcore_barrier()
        pltpu.sync_copy(acc_sh.at[pl.ds(base, stripe)], zb_v)
        pltpu.sync_copy(zb_v, out_hbm.at[c].at[pl.ds(base, stripe)])

    return deg


def _hop_kernel(n2, cp, nwin, second):
    stripe = n2 // NS
    rb = max(G * WIN, stripe)
    ngrp = nwin // G
    npairs = ngrp // 2
    cols = list(range(0, cp - 15, LANES))
    if cols[-1] + LANES < cp:
        cols.append(cp - LANES)

    @functools.partial(
        pl.kernel,
        out_type=jax.ShapeDtypeStruct((NC, n2, cp), F32),
        mesh=_sc_mesh(),
        scratch_types=[
            pltpu.VMEM((nwin, WIN), jnp.int32),
            pltpu.VMEM((nwin, WIN), jnp.int32),
            pltpu.VMEM((rb, cp), F32),
            pltpu.VMEM((rb, cp), F32),
            pltpu.VMEM((stripe,), F32),
            pltpu.VMEM_SHARED((n2, cp), F32),
            pltpu.VMEM_SHARED((n2, cp), F32),
            pltpu.SemaphoreType.DMA,
            pltpu.SemaphoreType.DMA,
            pltpu.SemaphoreType.DMA,
            pltpu.SemaphoreType.DMA,
        ],
        compiler_params=_SC_PARAMS,
    )
    def hop(h_hbm, dv_hbm, eip_hbm, out_hbm, src_v, dst_v, rows_a,
            rows_b, dv_v, u_sh, acc_sh, gsa, gsb, ssa, ssb):
        c = lax.axis_index("c")
        s = lax.axis_index("s")
        wid = s * NC + c
        base = s * stripe

        if second:
            pltpu.async_copy(h_hbm.at[0].at[pl.ds(base, stripe), :],
                             rows_a.at[pl.ds(0, stripe), :], gsa)
            pltpu.async_copy(h_hbm.at[1].at[pl.ds(base, stripe), :],
                             rows_b.at[pl.ds(0, stripe), :], gsb)
        else:
            pltpu.async_copy(h_hbm.at[pl.ds(base, stripe), :],
                             rows_a.at[pl.ds(0, stripe), :], gsa)
        pltpu.sync_copy(eip_hbm.at[0].at[wid], src_v)
        pltpu.sync_copy(eip_hbm.at[1].at[wid], dst_v)
        pltpu.sync_copy(dv_hbm.at[pl.ds(base, stripe)], dv_v)

        if second:
            pltpu.make_async_copy(h_hbm.at[0].at[pl.ds(base, stripe), :],
                                  rows_a.at[pl.ds(0, stripe), :], gsa).wait()
            pltpu.make_async_copy(h_hbm.at[1].at[pl.ds(base, stripe), :],
                                  rows_b.at[pl.ds(0, stripe), :], gsb).wait()
        else:
            pltpu.make_async_copy(h_hbm.at[pl.ds(base, stripe), :],
                                  rows_a.at[pl.ds(0, stripe), :], gsa).wait()

        @pl.loop(0, stripe, step=2)
        def _(r):
            for rr in range(2):
                vs = plsc.load_gather(
                    dv_v, [jnp.full((LANES,), r + rr, jnp.int32)])
                va = [rows_a[r + rr, pl.ds(c0, LANES)] for c0 in cols]
                if second:
                    va = [v + rows_b[r + rr, pl.ds(c0, LANES)]
                          for v, c0 in zip(va, cols)]
                for v, c0 in zip(va, cols):
                    rows_a[r + rr, pl.ds(c0, LANES)] = v * vs

        pltpu.async_copy(rows_a.at[pl.ds(0, stripe), :],
                         u_sh.at[pl.ds(base, stripe), :], gsa)

        @pl.when(c == 0)
        def _():
            pltpu.async_copy(rows_a.at[pl.ds(0, stripe), :],
                             acc_sh.at[pl.ds(base, stripe), :], gsb)

        @pl.when(c != 0)
        def _():
            @pl.loop(0, stripe)
            def _(r):
                for c0 in cols:
                    rows_b[r, pl.ds(c0, LANES)] = jnp.zeros((LANES,), F32)

            pltpu.async_copy(rows_b.at[pl.ds(0, stripe), :],
                             acc_sh.at[pl.ds(base, stripe), :], gsb)

        pltpu.make_async_copy(rows_a.at[pl.ds(0, stripe), :],
                              u_sh.at[pl.ds(base, stripe), :], gsa).wait()
        pltpu.make_async_copy(rows_a.at[pl.ds(0, stripe), :],
                              acc_sh.at[pl.ds(base, stripe), :], gsb).wait()

        def grp_gather(buf, sem, g):
            for k in range(G):
                pltpu.async_copy(u_sh.at[src_v.at[g * G + k]],
                                 buf.at[pl.ds(k * WIN, WIN), :], sem)

        def grp_gather_wait(buf, sem, g):
            for k in range(G):
                pltpu.make_async_copy(u_sh.at[src_v.at[g * G + k]],
                                      buf.at[pl.ds(k * WIN, WIN), :],
                                      sem).wait()

        def grp_scatter(buf, sem, g):
            for k in range(G):
                pltpu.async_copy(buf.at[pl.ds(k * WIN, WIN), :],
                                 acc_sh.at[dst_v.at[g * G + k]], sem,
                                 add=True)

        def grp_scatter_wait(buf, sem, g):
            for k in range(G):
                pltpu.make_async_copy(buf.at[pl.ds(k * WIN, WIN), :],
                                      acc_sh.at[dst_v.at[g * G + k]],
                                      sem).wait()

        plsc.subcore_barrier()
        grp_gather(rows_a, gsa, 0)

        @pl.loop(0, npairs)
        def _(it):
            g = it * 2
            grp_gather(rows_b, gsb, g + 1)
            grp_gather_wait(rows_a, gsa, g)
            grp_scatter(rows_a, ssa, g)
            grp_scatter_wait(rows_a, ssa, g)

            @pl.when(it + 1 < npairs)
            def _():
                grp_gather(rows_a, gsa, g + 2)

            grp_gather_wait(rows_b, gsb, g + 1)
            grp_scatter(rows_b, ssb, g + 1)
            grp_scatter_wait(rows_b, ssb, g + 1)

        plsc.subcore_barrier()
        pltpu.sync_copy(acc_sh.at[pl.ds(base, stripe), :],
                        out_hbm.at[c].at[pl.ds(base, stripe), :])

    return hop


def _matmul(x, w48, n, n2, cp):
    def body(x_ref, w_ref, o_ref):
        o_ref[0:n, :] = jnp.dot(x_ref[...], w_ref[...],
                                preferred_element_type=F32)
        o_ref[n:n2, :] = jnp.zeros((n2 - n, cp), F32)

    return pl.pallas_call(
        body, out_shape=jax.ShapeDtypeStruct((n2, cp), F32))(x, w48)


def _dv_kernel(degp, n1):
    def body(degp_ref, dv1_ref, dv2_ref):
        deg = degp_ref[0:1, :] + degp_ref[1:2, :] + 1.0
        dv1_ref[...] = lax.rsqrt(deg).reshape(n1)
        dv2_ref[...] = (1.0 / deg).reshape(n1)

    return pl.pallas_call(
        body,
        out_shape=(jax.ShapeDtypeStruct((n1,), F32),
                   jax.ShapeDtypeStruct((n1,), F32)))(degp)


def _finalize(h2, dv1, b2, n, c):
    def body(h2_ref, dv1_ref, b_ref, o_ref):
        dinv = dv1_ref[0:n].reshape(n, 1)
        h = (h2_ref[0, 0:n, :] + h2_ref[1, 0:n, :]) * dinv
        logits = h[:, :c] + b_ref[...]
        m = jnp.max(logits, axis=1, keepdims=True)
        e = jnp.exp(logits - m)
        lse = jnp.log(jnp.sum(e, axis=1, keepdims=True)) + m
        o_ref[...] = logits - lse

    return pl.pallas_call(
        body, out_shape=jax.ShapeDtypeStruct((n, c), F32))(h2, dv1, b2)


def kernel(x, edge_index, W, b):
    n, d = x.shape
    e = edge_index.shape[1]
    c = W.shape[1]
    cp = _ceil_to(c, 8)

    ew = _ceil_to(-(-e // NW), 2 * G * WIN)
    nwin = ew // WIN
    ep = NW * ew
    n1 = _ceil_to(n + 16, NS * LANES)
    n2 = _ceil_to(n + 16, NS * 8)

    pad = ep - e
    pad2 = jnp.concatenate(
        [jnp.zeros((1, pad), jnp.int32),
         (n + (jnp.arange(pad, dtype=jnp.int32) % 16)).reshape(1, pad)],
        axis=0)
    eip = jnp.concatenate([edge_index, pad2], axis=1).reshape(
        2, NW, nwin, WIN)

    w48 = jnp.pad(W, ((0, 0), (0, cp - c)))
    b2 = b.reshape(1, c)

    degp = _deg_kernel(n1, nwin)(eip)
    y = _matmul(x, w48, n, n2, cp)
    dv1, dv2 = _dv_kernel(degp, n1)

    h1 = _hop_kernel(n2, cp, nwin, second=False)(y, dv1, eip)
    h2 = _hop_kernel(n2, cp, nwin, second=True)(h1, dv2, eip)
    return _finalize(h2, dv1, b2, n, c)
